# baseline (speedup 1.0000x reference)
import jax
import jax.numpy as jnp
from jax import lax
from jax.experimental import pallas as pl
from jax.experimental.pallas import tpu as pltpu
from jax.experimental.pallas import tpu_sc as plsc

BATCH = 64
NUM_Q = 1000
NUM_CLS = 80
N = NUM_Q * NUM_CLS
K = 300
KPAD = 320
CAND = 384
CAP = 2048

NC, NS = 2, 16
NW = NC * NS
ROWS_PER_W = BATCH // NW

L1 = 11
S1 = 32 - 1 - L1 + 1
NB1 = 512
TOPBIN = 0x3F800000 >> S1
UNROLL = 4
HISTW = NB1 * 16 * UNROLL
PBU = 8
NV = N // 16


def _tec_body(scores_hbm, boxes_hbm, labels_hbm, oboxes_hbm, oscores_hbm,
              s_row, b_row, hist, cand_v, cand_i, gt_val, gt_idx, eq_idx,
              o_idx, o_scores, o_labels, o_boxes):
  wid = lax.axis_index("s") * NC + lax.axis_index("c")
  lanes = lax.iota(jnp.int32, 16)
  ones16 = jnp.ones((16,), jnp.int32)

  def scan_level(start_bin, k_rem, nslots=1):
    def cond(c):
      _, _, found = c
      return jnp.logical_not(found)

    def body(c):
      b, acc, _ = c
      cv = hist[pl.ds(b * 16, 16)]
      for u in range(1, nslots):
        cv = cv + hist[pl.ds(u * NB1 * 16 + b * 16, 16)]
      cnt = jnp.sum(cv)
      cross = acc + cnt >= k_rem
      return (jnp.where(cross, b, b - 1), jnp.where(cross, acc, acc + cnt),
              cross)

    b, acc, _ = lax.while_loop(
        cond, body, (start_bin + jnp.int32(0), jnp.int32(0), jnp.bool_(False)))
    return b, acc

  def clear_hist(nwords):
    z16 = jnp.zeros((16,), jnp.int32)

    def clr(i, _):
      for u in range(4):
        hist[pl.ds((i * 4 + u) * 16, 16)] = z16
      return 0
    lax.fori_loop(0, nwords // 64, clr, 0)

  def do_row(t, _):
    r = wid * ROWS_PER_W + t
    pltpu.sync_copy(scores_hbm.at[r], s_row)
    pltpu.sync_copy(boxes_hbm.at[r], b_row)

    clear_hist(HISTW)

    def h1(i, _):
      for u in range(UNROLL):
        k = plsc.bitcast(s_row[pl.ds((i * UNROLL + u) * 16, 16)], jnp.int32)
        plsc.addupdate_scatter(
            hist, [u * (NB1 * 16) + (k >> S1) * 16 + lanes], ones16)
      return 0
    lax.fori_loop(0, NV // UNROLL, h1, 0)
    b1, s1 = scan_level(jnp.int32(TOPBIN), jnp.int32(K), nslots=UNROLL)

    kmin = b1 << S1
    kminv = jnp.zeros((16,), jnp.int32) + kmin
    capv = jnp.zeros((16,), jnp.int32) + CAP

    def pb(i, p):
      vs, ms = [], []
      for u in range(PBU):
        v = s_row[pl.ds((i * PBU + u) * 16, 16)]
        vs.append(v)
        ms.append(plsc.bitcast(v, jnp.int32) >= kminv)
      anym = ms[0]
      for u in range(1, PBU):
        anym = jnp.logical_or(anym, ms[u])
      hit = jnp.any(anym)

      def slow(p2):
        for u in range(PBU):
          m = ms[u]
          mi = m.astype(jnp.int32)
          pos = p2 + plsc.cumsum(mi) - 1
          wm = jnp.logical_and(m, pos < capv)
          plsc.store_scatter(cand_v, [pos], vs[u], mask=wm)
          plsc.store_scatter(cand_i, [pos],
                             (i * PBU + u) * 16 + lanes, mask=wm)
          p2 = p2 + jnp.sum(mi)
        return p2

      return lax.cond(hit, slow, lambda p2: p2, p)

    n_cand = lax.fori_loop(0, NV // PBU, pb, jnp.int32(0))

    def fast_path(_):
      nv = (n_cand + 15) >> 4
      ncv = jnp.zeros((16,), jnp.int32) + n_cand

      nb4 = (n_cand + 63) >> 6

      def count_ge(q):
        qv = jnp.zeros((16,), jnp.int32) + q

        def cnt_body(j, a):
          for u in range(4):
            kc = plsc.bitcast(cand_v[pl.ds((j * 4 + u) * 16, 16)], jnp.int32)
            ok = jnp.logical_and(kc >= qv, (j * 4 + u) * 16 + lanes < ncv)
            a = a + ok.astype(jnp.int32)
          return a

        return jnp.sum(lax.fori_loop(0, nb4, cnt_body,
                                     jnp.zeros((16,), jnp.int32)))

      def bit_body(bi, p0):
        q = p0 | (jnp.int32(1) << (S1 - 1 - bi))
        return jnp.where(count_ge(q) >= K, q, p0)

      tk = lax.fori_loop(0, S1, bit_body, kmin)
      tkv = jnp.zeros((16,), jnp.int32) + tk
      n_gt_f = count_ge(tk + 1)
      n_eq_f = K - n_gt_f

      def cls(j, c):
        gp, ep = c
        v = cand_v[pl.ds(j * 16, 16)]
        kc = plsc.bitcast(v, jnp.int32)
        valid = j * 16 + lanes < ncv
        mgt = jnp.logical_and(kc > tkv, valid)
        meq = jnp.logical_and(kc == tkv, valid)
        idxv = cand_i[pl.ds(j * 16, 16)]
        pg = gp + plsc.cumsum(mgt.astype(jnp.int32)) - 1
        pe = ep + plsc.cumsum(meq.astype(jnp.int32)) - 1
        plsc.store_scatter(gt_val, [pg], v, mask=mgt)
        plsc.store_scatter(gt_idx, [pg], idxv, mask=mgt)
        plsc.store_scatter(eq_idx, [pe], idxv,
                           mask=jnp.logical_and(meq, pe < n_eq_f))
        return (gp + jnp.sum(mgt.astype(jnp.int32)),
                ep + jnp.sum(meq.astype(jnp.int32)))

      lax.fori_loop(0, nv, cls, (jnp.int32(0), jnp.int32(0)))
      return tk, n_gt_f

    def slow_path(_):
      def refine(shift, nbins, mshift, mval, k_rem):
        clear_hist(nbins * 16)

        def h(i, _):
          k = plsc.bitcast(s_row[pl.ds(i * 16, 16)], jnp.int32)
          m = (k >> mshift) == mval
          d = ((k >> shift) & (nbins - 1)) * 16 + lanes

          def add(_):
            plsc.addupdate_scatter(hist, [d], ones16, mask=m)
            return 0
          lax.cond(jnp.any(m), add, lambda _: 0, 0)
          return 0
        lax.fori_loop(0, NV, h, 0)
        return scan_level(jnp.int32(nbins - 1), k_rem)

      b2, s2 = refine(12, 512, S1, b1, K - s1)
      b3, s3 = refine(3, 512, 12, (b1 << 9) | b2, K - s1 - s2)
      b4, s4 = refine(0, 8, 3, (((b1 << 9) | b2) << 9) | b3, K - s1 - s2 - s3)
      tk = (((((b1 << 9) | b2) << 9) | b3) << 3) | b4
      tkv = jnp.zeros((16,), jnp.int32) + tk
      n_gt = s1 + s2 + s3 + s4
      n_eq = K - n_gt

      def collect(i, c):
        gp, ep = c
        v = s_row[pl.ds(i * 16, 16)]
        kc = plsc.bitcast(v, jnp.int32)
        mgt = kc > tkv
        meq = kc == tkv

        def slow(c2):
          gp2, ep2 = c2
          idxv = i * 16 + lanes
          pg = gp2 + plsc.cumsum(mgt.astype(jnp.int32)) - 1
          pe = ep2 + plsc.cumsum(meq.astype(jnp.int32)) - 1
          plsc.store_scatter(gt_val, [pg], v, mask=mgt)
          plsc.store_scatter(gt_idx, [pg], idxv, mask=mgt)
          plsc.store_scatter(eq_idx, [pe], idxv,
                             mask=jnp.logical_and(meq, pe < n_eq))
          return (gp2 + jnp.sum(mgt.astype(jnp.int32)),
                  ep2 + jnp.sum(meq.astype(jnp.int32)))

        return lax.cond(jnp.any(jnp.logical_or(mgt, meq)), slow,
                        lambda c2: c2, (gp, ep))

      lax.fori_loop(0, NV, collect, (jnp.int32(0), jnp.int32(0)))
      return tk, n_gt

    def pre(i, _):
      gt_val[pl.ds(i * 16, 16)] = jnp.full((16,), -1.0, jnp.float32)
      gt_idx[pl.ds(i * 16, 16)] = jnp.full((16,), 1 << 24, jnp.int32)
      return 0
    lax.fori_loop(0, CAND // 16, pre, 0)

    def pre2(i, _):
      o_idx[pl.ds(i * 16, 16)] = jnp.zeros((16,), jnp.int32)
      return 0
    lax.fori_loop(0, KPAD // 16, pre2, 0)

    tk, n_gt = lax.cond(n_cand <= CAP, fast_path, slow_path, 0)
    n_eq = K - n_gt
    tkv = jnp.zeros((16,), jnp.int32) + tk

    ngb = (n_gt + 63) >> 6

    def rank_body(i, _):
      vib = jnp.zeros((16,), jnp.float32) + gt_val[pl.ds(i, 16)][0]
      iib = jnp.zeros((16,), jnp.int32) + gt_idx[pl.ds(i, 16)][0]

      def inner(j, a):
        for u in range(4):
          vj = gt_val[pl.ds((j * 4 + u) * 16, 16)]
          ij = gt_idx[pl.ds((j * 4 + u) * 16, 16)]
          beats = jnp.logical_or(vj > vib,
                                 jnp.logical_and(vj == vib, ij < iib))
          a = a + beats.astype(jnp.int32)
        return a

      acc = lax.fori_loop(0, ngb, inner, jnp.zeros((16,), jnp.int32))
      rkv = jnp.zeros((16,), jnp.int32) + jnp.sum(acc)
      m0 = lanes == 0
      plsc.store_scatter(o_scores, [rkv], vib, mask=m0)
      plsc.store_scatter(o_idx, [rkv], iib, mask=m0)
      return 0
    lax.fori_loop(0, n_gt, rank_body, 0)

    tf = plsc.bitcast(tkv, jnp.float32)

    def eqf(j, _):
      ev = eq_idx[pl.ds(j * 16, 16)]
      ln = j * 16 + lanes
      m = ln < n_eq
      pos = n_gt + ln
      plsc.store_scatter(o_idx, [pos], ev, mask=m)
      plsc.store_scatter(o_scores, [pos], tf, mask=m)
      return 0
    lax.fori_loop(0, (n_eq + 15) >> 4, eqf, 0)

    def post(j, _):
      idxv = o_idx[pl.ds(j * 16, 16)]
      q = ((idxv >> 4) * 13108) >> 16
      o_labels[pl.ds(j * 16, 16)] = idxv - q * NUM_CLS
      base = q * 4
      cx = plsc.load_gather(b_row, [base])
      cy = plsc.load_gather(b_row, [base + 1])
      w = plsc.load_gather(b_row, [base + 2])
      h = plsc.load_gather(b_row, [base + 3])
      ob = (j * 16 + lanes) * 4
      plsc.store_scatter(o_boxes, [ob], (cx - 0.5 * w) * 640.0)
      plsc.store_scatter(o_boxes, [ob + 1], (cy - 0.5 * h) * 640.0)
      plsc.store_scatter(o_boxes, [ob + 2], (cx + 0.5 * w) * 640.0)
      plsc.store_scatter(o_boxes, [ob + 3], (cy + 0.5 * h) * 640.0)
      return 0
    lax.fori_loop(0, KPAD // 16, post, 0)

    pltpu.sync_copy(o_labels, labels_hbm.at[r])
    pltpu.sync_copy(o_boxes, oboxes_hbm.at[r])
    pltpu.sync_copy(o_scores, oscores_hbm.at[r])
    return 0

  lax.fori_loop(0, ROWS_PER_W, do_row, 0)


_sc_topk = pl.kernel(
    _tec_body,
    out_type=(
        jax.ShapeDtypeStruct((BATCH, KPAD), jnp.int32),
        jax.ShapeDtypeStruct((BATCH, KPAD * 4), jnp.float32),
        jax.ShapeDtypeStruct((BATCH, KPAD), jnp.float32),
    ),
    mesh=plsc.VectorSubcoreMesh(core_axis_name="c", subcore_axis_name="s",
                                num_cores=NC, num_subcores=NS),
    compiler_params=pltpu.CompilerParams(needs_layout_passes=False),
    scratch_types=[
        pltpu.VMEM((N,), jnp.float32),
        pltpu.VMEM((NUM_Q * 4,), jnp.float32),
        pltpu.VMEM((HISTW,), jnp.int32),
        pltpu.VMEM((CAP,), jnp.float32),
        pltpu.VMEM((CAP,), jnp.int32),
        pltpu.VMEM((CAND,), jnp.float32),
        pltpu.VMEM((CAND,), jnp.int32),
        pltpu.VMEM((CAND,), jnp.int32),
        pltpu.VMEM((KPAD,), jnp.int32),
        pltpu.VMEM((KPAD,), jnp.float32),
        pltpu.VMEM((KPAD,), jnp.int32),
        pltpu.VMEM((KPAD * 4,), jnp.float32),
    ],
)


def kernel(samples, pred_logits, pred_boxes):
  del samples
  scores = jax.nn.sigmoid(pred_logits).reshape(BATCH, N)
  boxes_flat = pred_boxes.reshape(BATCH, NUM_Q * 4)
  labels, boxes, top_scores = _sc_topk(scores, boxes_flat)
  return (labels[:, :K], boxes.reshape(BATCH, KPAD, 4)[:, :K, :],
          top_scores[:, :K])


# speedup vs baseline: 3.8269x; 3.8269x over previous
import jax
import jax.numpy as jnp
from jax import lax
from jax.experimental import pallas as pl
from jax.experimental.pallas import tpu as pltpu
from jax.experimental.pallas import tpu_sc as plsc

BATCH = 64
NUM_Q = 1000
NUM_CLS = 80
N = NUM_Q * NUM_CLS
K = 300
KPAD = 320
CAND = 384
CAP = 4096

NC, NS = 2, 16
NW = NC * NS
ROWS_PER_W = BATCH // NW

L1 = 11
S1 = 32 - 1 - L1 + 1
NB1 = 512
TOPBIN = 0x3F800000 >> S1
NSLOT = 2
UNROLL = 4
HISTW = NB1 * 16 * NSLOT
PBU = 4
NV = N // 16


def _tec_body(scores_hbm, boxes_hbm, labels_hbm, oboxes_hbm, oscores_hbm,
              s_row, b_row, hist, cand_v, cand_i, gt_val, gt_idx, eq_idx,
              o_idx, o_scores, o_labels, o_boxes):
  wid = lax.axis_index("s") * NC + lax.axis_index("c")
  lanes = lax.iota(jnp.int32, 16)
  ones16 = jnp.ones((16,), jnp.int32)

  def scan_level(start_bin, k_rem, nslots=1):
    def cond(c):
      _, _, found = c
      return jnp.logical_not(found)

    def body(c):
      b, acc, _ = c
      cv = hist[pl.ds(b * 16, 16)]
      for u in range(1, nslots):
        cv = cv + hist[pl.ds(u * NB1 * 16 + b * 16, 16)]
      cnt = jnp.sum(cv)
      cross = acc + cnt >= k_rem
      return (jnp.where(cross, b, b - 1), jnp.where(cross, acc, acc + cnt),
              cross)

    b, acc, _ = lax.while_loop(
        cond, body, (start_bin + jnp.int32(0), jnp.int32(0), jnp.bool_(False)))
    return b, acc

  def clear_hist(nwords):
    z16 = jnp.zeros((16,), jnp.int32)

    def clr(i, _):
      for u in range(4):
        hist[pl.ds((i * 4 + u) * 16, 16)] = z16
      return 0
    lax.fori_loop(0, nwords // 64, clr, 0)

  def do_row(t, _):
    r = wid * ROWS_PER_W + t
    pltpu.sync_copy(scores_hbm.at[r], s_row)
    pltpu.sync_copy(boxes_hbm.at[r], b_row)

    clear_hist(HISTW)

    def h1(i, _):
      for u in range(UNROLL):
        k = plsc.bitcast(s_row[pl.ds((i * UNROLL + u) * 16, 16)], jnp.int32)
        plsc.addupdate_scatter(
            hist, [(u % NSLOT) * (NB1 * 16) + (k >> S1) * 16 + lanes], ones16)
      return 0
    lax.fori_loop(0, NV // UNROLL, h1, 0)
    b1, s1 = scan_level(jnp.int32(TOPBIN), jnp.int32(K), nslots=NSLOT)

    kmin = b1 << S1
    kminv = jnp.zeros((16,), jnp.int32) + kmin
    capv = jnp.zeros((16,), jnp.int32) + CAP

    def pb(i, p):
      vs, ms = [], []
      for u in range(PBU):
        v = s_row[pl.ds((i * PBU + u) * 16, 16)]
        vs.append(v)
        ms.append(plsc.bitcast(v, jnp.int32) >= kminv)
      anym = ms[0]
      for u in range(1, PBU):
        anym = jnp.logical_or(anym, ms[u])
      hit = jnp.any(anym)

      def slow(p2):
        cnts = [plsc.all_reduce_population_count(ms[u]) for u in range(PBU)]
        base = jnp.zeros((16,), jnp.int32) + p2
        for u in range(PBU):
          m = ms[u]
          pos = base + plsc.cumsum(m.astype(jnp.int32)) - 1
          wm = jnp.logical_and(m, pos < capv)
          plsc.store_scatter(cand_v, [pos], vs[u], mask=wm)
          plsc.store_scatter(cand_i, [pos],
                             (i * PBU + u) * 16 + lanes, mask=wm)
          base = base + cnts[u]
        return base[0]

      return lax.cond(hit, slow, lambda p2: p2, p)

    n_cand = lax.fori_loop(0, NV // PBU, pb, jnp.int32(0))

    def fast_path(_):
      nv = (n_cand + 15) >> 4
      ncv = jnp.zeros((16,), jnp.int32) + n_cand

      nb4 = (n_cand + 63) >> 6

      def count_ge(q):
        qv = jnp.zeros((16,), jnp.int32) + q

        def cnt_body(j, a):
          for u in range(4):
            kc = plsc.bitcast(cand_v[pl.ds((j * 4 + u) * 16, 16)], jnp.int32)
            ok = jnp.logical_and(kc >= qv, (j * 4 + u) * 16 + lanes < ncv)
            a = a + ok.astype(jnp.int32)
          return a

        return jnp.sum(lax.fori_loop(0, nb4, cnt_body,
                                     jnp.zeros((16,), jnp.int32)))

      def bit_body(bi, p0):
        q = p0 | (jnp.int32(1) << (S1 - 1 - bi))
        return jnp.where(count_ge(q) >= K, q, p0)

      tk = lax.fori_loop(0, S1, bit_body, kmin)
      tkv = jnp.zeros((16,), jnp.int32) + tk
      n_gt_f = count_ge(tk + 1)
      n_eq_f = K - n_gt_f

      def cls(j, c):
        gp, ep = c
        v = cand_v[pl.ds(j * 16, 16)]
        kc = plsc.bitcast(v, jnp.int32)
        valid = j * 16 + lanes < ncv
        mgt = jnp.logical_and(kc > tkv, valid)
        meq = jnp.logical_and(kc == tkv, valid)
        idxv = cand_i[pl.ds(j * 16, 16)]
        pg = gp + plsc.cumsum(mgt.astype(jnp.int32)) - 1
        pe = ep + plsc.cumsum(meq.astype(jnp.int32)) - 1
        plsc.store_scatter(gt_val, [pg], v, mask=mgt)
        plsc.store_scatter(gt_idx, [pg], idxv, mask=mgt)
        plsc.store_scatter(eq_idx, [pe], idxv,
                           mask=jnp.logical_and(meq, pe < n_eq_f))
        return (gp + jnp.sum(mgt.astype(jnp.int32)),
                ep + jnp.sum(meq.astype(jnp.int32)))

      lax.fori_loop(0, nv, cls, (jnp.int32(0), jnp.int32(0)))
      return tk, n_gt_f

    def slow_path(_):
      def refine(shift, nbins, mshift, mval, k_rem):
        clear_hist(nbins * 16)

        def h(i, _):
          k = plsc.bitcast(s_row[pl.ds(i * 16, 16)], jnp.int32)
          m = (k >> mshift) == mval
          d = ((k >> shift) & (nbins - 1)) * 16 + lanes

          def add(_):
            plsc.addupdate_scatter(hist, [d], ones16, mask=m)
            return 0
          lax.cond(jnp.any(m), add, lambda _: 0, 0)
          return 0
        lax.fori_loop(0, NV, h, 0)
        return scan_level(jnp.int32(nbins - 1), k_rem)

      b2, s2 = refine(12, 512, S1, b1, K - s1)
      b3, s3 = refine(3, 512, 12, (b1 << 9) | b2, K - s1 - s2)
      b4, s4 = refine(0, 8, 3, (((b1 << 9) | b2) << 9) | b3, K - s1 - s2 - s3)
      tk = (((((b1 << 9) | b2) << 9) | b3) << 3) | b4
      tkv = jnp.zeros((16,), jnp.int32) + tk
      n_gt = s1 + s2 + s3 + s4
      n_eq = K - n_gt

      def collect(i, c):
        gp, ep = c
        v = s_row[pl.ds(i * 16, 16)]
        kc = plsc.bitcast(v, jnp.int32)
        mgt = kc > tkv
        meq = kc == tkv

        def slow(c2):
          gp2, ep2 = c2
          idxv = i * 16 + lanes
          pg = gp2 + plsc.cumsum(mgt.astype(jnp.int32)) - 1
          pe = ep2 + plsc.cumsum(meq.astype(jnp.int32)) - 1
          plsc.store_scatter(gt_val, [pg], v, mask=mgt)
          plsc.store_scatter(gt_idx, [pg], idxv, mask=mgt)
          plsc.store_scatter(eq_idx, [pe], idxv,
                             mask=jnp.logical_and(meq, pe < n_eq))
          return (gp2 + jnp.sum(mgt.astype(jnp.int32)),
                  ep2 + jnp.sum(meq.astype(jnp.int32)))

        return lax.cond(jnp.any(jnp.logical_or(mgt, meq)), slow,
                        lambda c2: c2, (gp, ep))

      lax.fori_loop(0, NV, collect, (jnp.int32(0), jnp.int32(0)))
      return tk, n_gt

    def pre(i, _):
      gt_val[pl.ds(i * 16, 16)] = jnp.full((16,), -1.0, jnp.float32)
      gt_idx[pl.ds(i * 16, 16)] = jnp.full((16,), 1 << 24, jnp.int32)
      return 0
    lax.fori_loop(0, CAND // 16, pre, 0)

    def pre2(i, _):
      o_idx[pl.ds(i * 16, 16)] = jnp.zeros((16,), jnp.int32)
      return 0
    lax.fori_loop(0, KPAD // 16, pre2, 0)

    tk, n_gt = lax.cond(n_cand <= CAP, fast_path, slow_path, 0)
    n_eq = K - n_gt
    tkv = jnp.zeros((16,), jnp.int32) + tk

    ngb = (n_gt + 63) >> 6

    def rank_body(i, _):
      vib = jnp.zeros((16,), jnp.float32) + gt_val[pl.ds(i, 16)][0]
      iib = jnp.zeros((16,), jnp.int32) + gt_idx[pl.ds(i, 16)][0]

      def inner(j, a):
        for u in range(4):
          vj = gt_val[pl.ds((j * 4 + u) * 16, 16)]
          ij = gt_idx[pl.ds((j * 4 + u) * 16, 16)]
          beats = jnp.logical_or(vj > vib,
                                 jnp.logical_and(vj == vib, ij < iib))
          a = a + beats.astype(jnp.int32)
        return a

      acc = lax.fori_loop(0, ngb, inner, jnp.zeros((16,), jnp.int32))
      rkv = jnp.zeros((16,), jnp.int32) + jnp.sum(acc)
      m0 = lanes == 0
      plsc.store_scatter(o_scores, [rkv], vib, mask=m0)
      plsc.store_scatter(o_idx, [rkv], iib, mask=m0)
      return 0
    lax.fori_loop(0, n_gt, rank_body, 0)

    tf = plsc.bitcast(tkv, jnp.float32)

    def eqf(j, _):
      ev = eq_idx[pl.ds(j * 16, 16)]
      ln = j * 16 + lanes
      m = ln < n_eq
      pos = n_gt + ln
      plsc.store_scatter(o_idx, [pos], ev, mask=m)
      plsc.store_scatter(o_scores, [pos], tf, mask=m)
      return 0
    lax.fori_loop(0, (n_eq + 15) >> 4, eqf, 0)

    def post(j, _):
      idxv = o_idx[pl.ds(j * 16, 16)]
      q = ((idxv >> 4) * 13108) >> 16
      o_labels[pl.ds(j * 16, 16)] = idxv - q * NUM_CLS
      base = q * 4
      cx = plsc.load_gather(b_row, [base])
      cy = plsc.load_gather(b_row, [base + 1])
      w = plsc.load_gather(b_row, [base + 2])
      h = plsc.load_gather(b_row, [base + 3])
      ob = (j * 16 + lanes) * 4
      plsc.store_scatter(o_boxes, [ob], (cx - 0.5 * w) * 640.0)
      plsc.store_scatter(o_boxes, [ob + 1], (cy - 0.5 * h) * 640.0)
      plsc.store_scatter(o_boxes, [ob + 2], (cx + 0.5 * w) * 640.0)
      plsc.store_scatter(o_boxes, [ob + 3], (cy + 0.5 * h) * 640.0)
      return 0
    lax.fori_loop(0, KPAD // 16, post, 0)

    pltpu.sync_copy(o_labels, labels_hbm.at[r])
    pltpu.sync_copy(o_boxes, oboxes_hbm.at[r])
    pltpu.sync_copy(o_scores, oscores_hbm.at[r])
    return 0

  lax.fori_loop(0, ROWS_PER_W, do_row, 0)


_sc_topk = pl.kernel(
    _tec_body,
    out_type=(
        jax.ShapeDtypeStruct((BATCH, KPAD), jnp.int32),
        jax.ShapeDtypeStruct((BATCH, KPAD * 4), jnp.float32),
        jax.ShapeDtypeStruct((BATCH, KPAD), jnp.float32),
    ),
    mesh=plsc.VectorSubcoreMesh(core_axis_name="c", subcore_axis_name="s",
                                num_cores=NC, num_subcores=NS),
    compiler_params=pltpu.CompilerParams(needs_layout_passes=False),
    scratch_types=[
        pltpu.VMEM((N,), jnp.float32),
        pltpu.VMEM((NUM_Q * 4,), jnp.float32),
        pltpu.VMEM((HISTW,), jnp.int32),
        pltpu.VMEM((CAP,), jnp.float32),
        pltpu.VMEM((CAP,), jnp.int32),
        pltpu.VMEM((CAND,), jnp.float32),
        pltpu.VMEM((CAND,), jnp.int32),
        pltpu.VMEM((CAND,), jnp.int32),
        pltpu.VMEM((KPAD,), jnp.int32),
        pltpu.VMEM((KPAD,), jnp.float32),
        pltpu.VMEM((KPAD,), jnp.int32),
        pltpu.VMEM((KPAD * 4,), jnp.float32),
    ],
)


def kernel(samples, pred_logits, pred_boxes):
  del samples
  scores = jax.nn.sigmoid(pred_logits).reshape(BATCH, N)
  boxes_flat = pred_boxes.reshape(BATCH, NUM_Q * 4)
  labels, boxes, top_scores = _sc_topk(scores, boxes_flat)
  return (labels[:, :K], boxes.reshape(BATCH, KPAD, 4)[:, :K, :],
          top_scores[:, :K])


# speedup vs baseline: 3.8272x; 1.0001x over previous
import jax
import jax.numpy as jnp
from jax import lax
from jax.experimental import pallas as pl
from jax.experimental.pallas import tpu as pltpu
from jax.experimental.pallas import tpu_sc as plsc

BATCH = 64
NUM_Q = 1000
NUM_CLS = 80
N = NUM_Q * NUM_CLS
K = 300
KPAD = 320
CAND = 384
CAP = 4096

NC, NS = 2, 16
NW = NC * NS
ROWS_PER_W = BATCH // NW

L1 = 11
S1 = 32 - 1 - L1 + 1
NB1 = 512
TOPBIN = 0x3F800000 >> S1
NSLOT = 2
UNROLL = 4
HISTW = NB1 * 16 * NSLOT
PBU = 4
NV = N // 16


def _tec_body(scores_hbm, boxes_hbm, labels_hbm, oboxes_hbm, oscores_hbm,
              s_row, b_row, hist, cand_v, cand_i, gt_val, gt_idx, eq_idx,
              o_idx, o_scores, o_labels, o_boxes):
  wid = lax.axis_index("s") * NC + lax.axis_index("c")
  lanes = lax.iota(jnp.int32, 16)
  ones16 = jnp.ones((16,), jnp.int32)

  def scan_level(start_bin, k_rem, nslots=1):
    def cond(c):
      _, _, found = c
      return jnp.logical_not(found)

    def body(c):
      b, acc, _ = c
      cv = hist[pl.ds(b * 16, 16)]
      for u in range(1, nslots):
        cv = cv + hist[pl.ds(u * NB1 * 16 + b * 16, 16)]
      cnt = jnp.sum(cv)
      cross = acc + cnt >= k_rem
      return (jnp.where(cross, b, b - 1), jnp.where(cross, acc, acc + cnt),
              cross)

    b, acc, _ = lax.while_loop(
        cond, body, (start_bin + jnp.int32(0), jnp.int32(0), jnp.bool_(False)))
    return b, acc

  def clear_hist(nwords):
    z16 = jnp.zeros((16,), jnp.int32)

    def clr(i, _):
      for u in range(4):
        hist[pl.ds((i * 4 + u) * 16, 16)] = z16
      return 0
    lax.fori_loop(0, nwords // 64, clr, 0)

  def do_row(t, _):
    r = wid * ROWS_PER_W + t
    pltpu.sync_copy(scores_hbm.at[r], s_row)
    pltpu.sync_copy(boxes_hbm.at[r], b_row)

    clear_hist(HISTW)

    def h1(i, _):
      for u in range(UNROLL):
        k = plsc.bitcast(s_row[pl.ds((i * UNROLL + u) * 16, 16)], jnp.int32)
        d = ((k >> (S1 - 4)) & ((NB1 - 1) << 4)) + lanes
        plsc.addupdate_scatter(hist, [(u % NSLOT) * (NB1 * 16) + d], ones16)
      return 0
    lax.fori_loop(0, NV // UNROLL, h1, 0)
    b1, s1 = scan_level(jnp.int32(TOPBIN), jnp.int32(K), nslots=NSLOT)

    kmin = b1 << S1
    kminv = jnp.zeros((16,), jnp.int32) + kmin
    capv = jnp.zeros((16,), jnp.int32) + CAP

    def pb(i, p):
      vs, ms = [], []
      for u in range(PBU):
        v = s_row[pl.ds((i * PBU + u) * 16, 16)]
        vs.append(v)
        ms.append(plsc.bitcast(v, jnp.int32) >= kminv)
      anym = ms[0]
      for u in range(1, PBU):
        anym = jnp.logical_or(anym, ms[u])
      hit = jnp.any(anym)

      def slow(p2):
        cnts = [plsc.all_reduce_population_count(ms[u]) for u in range(PBU)]
        base = jnp.zeros((16,), jnp.int32) + p2
        for u in range(PBU):
          m = ms[u]
          pos = base + plsc.cumsum(m.astype(jnp.int32)) - 1
          wm = jnp.logical_and(m, pos < capv)
          plsc.store_scatter(cand_v, [pos], vs[u], mask=wm)
          plsc.store_scatter(cand_i, [pos],
                             (i * PBU + u) * 16 + lanes, mask=wm)
          base = base + cnts[u]
        return base[0]

      return lax.cond(hit, slow, lambda p2: p2, p)

    n_cand = lax.fori_loop(0, NV // PBU, pb, jnp.int32(0))

    def fast_path(_):
      nv = (n_cand + 15) >> 4
      ncv = jnp.zeros((16,), jnp.int32) + n_cand

      nb4 = (n_cand + 63) >> 6

      def count_ge(q):
        qv = jnp.zeros((16,), jnp.int32) + q

        def cnt_body(j, a):
          for u in range(4):
            kc = plsc.bitcast(cand_v[pl.ds((j * 4 + u) * 16, 16)], jnp.int32)
            ok = jnp.logical_and(kc >= qv, (j * 4 + u) * 16 + lanes < ncv)
            a = a + ok.astype(jnp.int32)
          return a

        return jnp.sum(lax.fori_loop(0, nb4, cnt_body,
                                     jnp.zeros((16,), jnp.int32)))

      def bit_body(bi, p0):
        q = p0 | (jnp.int32(1) << (S1 - 1 - bi))
        return jnp.where(count_ge(q) >= K, q, p0)

      tk = lax.fori_loop(0, S1, bit_body, kmin)
      tkv = jnp.zeros((16,), jnp.int32) + tk
      n_gt_f = count_ge(tk + 1)
      n_eq_f = K - n_gt_f

      def cls(j, c):
        gp, ep = c
        v = cand_v[pl.ds(j * 16, 16)]
        kc = plsc.bitcast(v, jnp.int32)
        valid = j * 16 + lanes < ncv
        mgt = jnp.logical_and(kc > tkv, valid)
        meq = jnp.logical_and(kc == tkv, valid)
        idxv = cand_i[pl.ds(j * 16, 16)]
        pg = gp + plsc.cumsum(mgt.astype(jnp.int32)) - 1
        pe = ep + plsc.cumsum(meq.astype(jnp.int32)) - 1
        plsc.store_scatter(gt_val, [pg], v, mask=mgt)
        plsc.store_scatter(gt_idx, [pg], idxv, mask=mgt)
        plsc.store_scatter(eq_idx, [pe], idxv,
                           mask=jnp.logical_and(meq, pe < n_eq_f))
        return (gp + plsc.all_reduce_population_count(mgt)[0],
                ep + plsc.all_reduce_population_count(meq)[0])

      lax.fori_loop(0, nv, cls, (jnp.int32(0), jnp.int32(0)))
      return tk, n_gt_f

    def slow_path(_):
      def refine(shift, nbins, mshift, mval, k_rem):
        clear_hist(nbins * 16)

        def h(i, _):
          k = plsc.bitcast(s_row[pl.ds(i * 16, 16)], jnp.int32)
          m = (k >> mshift) == mval
          d = ((k >> shift) & (nbins - 1)) * 16 + lanes

          def add(_):
            plsc.addupdate_scatter(hist, [d], ones16, mask=m)
            return 0
          lax.cond(jnp.any(m), add, lambda _: 0, 0)
          return 0
        lax.fori_loop(0, NV, h, 0)
        return scan_level(jnp.int32(nbins - 1), k_rem)

      b2, s2 = refine(12, 512, S1, b1, K - s1)
      b3, s3 = refine(3, 512, 12, (b1 << 9) | b2, K - s1 - s2)
      b4, s4 = refine(0, 8, 3, (((b1 << 9) | b2) << 9) | b3, K - s1 - s2 - s3)
      tk = (((((b1 << 9) | b2) << 9) | b3) << 3) | b4
      tkv = jnp.zeros((16,), jnp.int32) + tk
      n_gt = s1 + s2 + s3 + s4
      n_eq = K - n_gt

      def collect(i, c):
        gp, ep = c
        v = s_row[pl.ds(i * 16, 16)]
        kc = plsc.bitcast(v, jnp.int32)
        mgt = kc > tkv
        meq = kc == tkv

        def slow(c2):
          gp2, ep2 = c2
          idxv = i * 16 + lanes
          pg = gp2 + plsc.cumsum(mgt.astype(jnp.int32)) - 1
          pe = ep2 + plsc.cumsum(meq.astype(jnp.int32)) - 1
          plsc.store_scatter(gt_val, [pg], v, mask=mgt)
          plsc.store_scatter(gt_idx, [pg], idxv, mask=mgt)
          plsc.store_scatter(eq_idx, [pe], idxv,
                             mask=jnp.logical_and(meq, pe < n_eq))
          return (gp2 + jnp.sum(mgt.astype(jnp.int32)),
                  ep2 + jnp.sum(meq.astype(jnp.int32)))

        return lax.cond(jnp.any(jnp.logical_or(mgt, meq)), slow,
                        lambda c2: c2, (gp, ep))

      lax.fori_loop(0, NV, collect, (jnp.int32(0), jnp.int32(0)))
      return tk, n_gt

    def pre(i, _):
      gt_val[pl.ds(i * 16, 16)] = jnp.full((16,), -1.0, jnp.float32)
      gt_idx[pl.ds(i * 16, 16)] = jnp.full((16,), 1 << 24, jnp.int32)
      return 0
    lax.fori_loop(0, CAND // 16, pre, 0)

    def pre2(i, _):
      o_idx[pl.ds(i * 16, 16)] = jnp.zeros((16,), jnp.int32)
      return 0
    lax.fori_loop(0, KPAD // 16, pre2, 0)

    tk, n_gt = lax.cond(n_cand <= CAP, fast_path, slow_path, 0)
    n_eq = K - n_gt
    tkv = jnp.zeros((16,), jnp.int32) + tk

    ngb = (n_gt + 63) >> 6

    def rank_body(i, _):
      vib = jnp.zeros((16,), jnp.float32) + gt_val[pl.ds(i, 16)][0]
      iib = jnp.zeros((16,), jnp.int32) + gt_idx[pl.ds(i, 16)][0]

      def inner(j, a):
        for u in range(4):
          vj = gt_val[pl.ds((j * 4 + u) * 16, 16)]
          ij = gt_idx[pl.ds((j * 4 + u) * 16, 16)]
          beats = jnp.logical_or(vj > vib,
                                 jnp.logical_and(vj == vib, ij < iib))
          a = a + beats.astype(jnp.int32)
        return a

      acc = lax.fori_loop(0, ngb, inner, jnp.zeros((16,), jnp.int32))
      rkv = jnp.zeros((16,), jnp.int32) + jnp.sum(acc)
      m0 = lanes == 0
      plsc.store_scatter(o_scores, [rkv], vib, mask=m0)
      plsc.store_scatter(o_idx, [rkv], iib, mask=m0)
      return 0
    lax.fori_loop(0, n_gt, rank_body, 0)

    tf = plsc.bitcast(tkv, jnp.float32)

    def eqf(j, _):
      ev = eq_idx[pl.ds(j * 16, 16)]
      ln = j * 16 + lanes
      m = ln < n_eq
      pos = n_gt + ln
      plsc.store_scatter(o_idx, [pos], ev, mask=m)
      plsc.store_scatter(o_scores, [pos], tf, mask=m)
      return 0
    lax.fori_loop(0, (n_eq + 15) >> 4, eqf, 0)

    def post(j, _):
      idxv = o_idx[pl.ds(j * 16, 16)]
      q = ((idxv >> 4) * 13108) >> 16
      o_labels[pl.ds(j * 16, 16)] = idxv - q * NUM_CLS
      base = q * 4
      cx = plsc.load_gather(b_row, [base])
      cy = plsc.load_gather(b_row, [base + 1])
      w = plsc.load_gather(b_row, [base + 2])
      h = plsc.load_gather(b_row, [base + 3])
      ob = (j * 16 + lanes) * 4
      plsc.store_scatter(o_boxes, [ob], (cx - 0.5 * w) * 640.0)
      plsc.store_scatter(o_boxes, [ob + 1], (cy - 0.5 * h) * 640.0)
      plsc.store_scatter(o_boxes, [ob + 2], (cx + 0.5 * w) * 640.0)
      plsc.store_scatter(o_boxes, [ob + 3], (cy + 0.5 * h) * 640.0)
      return 0
    lax.fori_loop(0, KPAD // 16, post, 0)

    pltpu.sync_copy(o_labels, labels_hbm.at[r])
    pltpu.sync_copy(o_boxes, oboxes_hbm.at[r])
    pltpu.sync_copy(o_scores, oscores_hbm.at[r])
    return 0

  lax.fori_loop(0, ROWS_PER_W, do_row, 0)


_sc_topk = pl.kernel(
    _tec_body,
    out_type=(
        jax.ShapeDtypeStruct((BATCH, KPAD), jnp.int32),
        jax.ShapeDtypeStruct((BATCH, KPAD * 4), jnp.float32),
        jax.ShapeDtypeStruct((BATCH, KPAD), jnp.float32),
    ),
    mesh=plsc.VectorSubcoreMesh(core_axis_name="c", subcore_axis_name="s",
                                num_cores=NC, num_subcores=NS),
    compiler_params=pltpu.CompilerParams(needs_layout_passes=False),
    scratch_types=[
        pltpu.VMEM((N,), jnp.float32),
        pltpu.VMEM((NUM_Q * 4,), jnp.float32),
        pltpu.VMEM((HISTW,), jnp.int32),
        pltpu.VMEM((CAP,), jnp.float32),
        pltpu.VMEM((CAP,), jnp.int32),
        pltpu.VMEM((CAND,), jnp.float32),
        pltpu.VMEM((CAND,), jnp.int32),
        pltpu.VMEM((CAND,), jnp.int32),
        pltpu.VMEM((KPAD,), jnp.int32),
        pltpu.VMEM((KPAD,), jnp.float32),
        pltpu.VMEM((KPAD,), jnp.int32),
        pltpu.VMEM((KPAD * 4,), jnp.float32),
    ],
)


def kernel(samples, pred_logits, pred_boxes):
  del samples
  scores = jax.nn.sigmoid(pred_logits).reshape(BATCH, N)
  boxes_flat = pred_boxes.reshape(BATCH, NUM_Q * 4)
  labels, boxes, top_scores = _sc_topk(scores, boxes_flat)
  return (labels[:, :K], boxes.reshape(BATCH, KPAD, 4)[:, :K, :],
          top_scores[:, :K])
